# baseline (speedup 1.0000x reference)
import functools

import jax
import jax.numpy as jnp
from jax import lax
from jax.experimental import pallas as pl
from jax.experimental.pallas import tpu as pltpu
from jax.experimental.pallas import tpu_sc as plsc

_A = 16
_NA = 32
_B = 256
_N = _B * _A
_OBS = 128
_NW = 32
_GPW = _B // _NW

_NOISE_CACHE = None


def _noise_const():
    global _NOISE_CACHE
    if _NOISE_CACHE is None:
        def build():
            nz = jax.random.normal(
                jax.random.key(1), (_N, _A, _NA), dtype=jnp.float32) * 0.1
            return nz.reshape(_B, _A * _A, _NA)
        try:
            with jax.ensure_compile_time_eval():
                _NOISE_CACHE = build()
        except Exception:
            return build()
    return _NOISE_CACHE


def _sc_body(w_hbm, pol_hbm, act_hbm, obs_hbm, noise_hbm, out_hbm,
             wv, pol_v, act_v, obs_v, noise_v, z2_v, sem_in, sem_out):
    cid = lax.axis_index("c")
    sid = lax.axis_index("s")
    wid = sid * 2 + cid
    pltpu.sync_copy(w_hbm, wv)
    wvec = wv[...]
    onemw = 1.0 - wvec
    inv = 1.0 / _A

    def group_body(g, carry):
        b = wid * _GPW + g
        r0 = b * _A
        hin = [
            pltpu.async_copy(pol_hbm.at[pl.ds(r0, _A), :], pol_v, sem_in),
            pltpu.async_copy(act_hbm.at[pl.ds(r0, _A), :], act_v, sem_in),
            pltpu.async_copy(obs_hbm.at[pl.ds(r0, _A), :], obs_v, sem_in),
            pltpu.async_copy(noise_hbm.at[b], noise_v, sem_in),
        ]
        for h in hin:
            h.wait()

        hout = []
        for i in range(_A):
            hout.append(pltpu.async_copy(
                obs_v, out_hbm.at[r0 + i, :, pl.ds(0, _OBS)], sem_out))

        for half in range(2):
            cs = pl.ds(half * 16, 16)
            sm = jnp.zeros((16,), jnp.float32)
            diffs = []
            for k in range(_A):
                pv = pol_v[k, cs]
                av = act_v[k, cs]
                sm = sm + (wvec * av + onemw * pv)
                diffs.append(wvec * (pv - av))
            e2s = [d + sm for d in diffs]
            for i in range(_A):
                nvs = [noise_v[i * _A + j, cs] for j in range(_A)]
                sn = nvs[0]
                for j in range(1, _A):
                    sn = sn + nvs[j]
                for k in range(_A):
                    z2_v[i, k, cs] = (e2s[k] + sn - nvs[k]) * inv

        hout.append(pltpu.async_copy(
            z2_v, out_hbm.at[pl.ds(r0, _A), :, pl.ds(_OBS, _NA)], sem_out))
        for h in hout:
            h.wait()
        return carry

    lax.fori_loop(0, _GPW, group_body, 0)


@functools.partial(
    pl.kernel,
    out_type=jax.ShapeDtypeStruct((_N, _A, _OBS + _NA), jnp.float32),
    mesh=plsc.VectorSubcoreMesh(core_axis_name="c", subcore_axis_name="s"),
    scratch_types=[
        pltpu.VMEM((16,), jnp.float32),
        pltpu.VMEM((_A, _NA), jnp.float32),
        pltpu.VMEM((_A, _NA), jnp.float32),
        pltpu.VMEM((_A, _OBS), jnp.float32),
        pltpu.VMEM((_A * _A, _NA), jnp.float32),
        pltpu.VMEM((_A, _A, _NA), jnp.float32),
        pltpu.SemaphoreType.DMA,
        pltpu.SemaphoreType.DMA,
    ],
)
def _sc_run(w_hbm, pol_hbm, act_hbm, obs_hbm, noise_hbm, out_hbm,
            wv, pol_v, act_v, obs_v, noise_v, z2_v, sem_in, sem_out):
    _sc_body(w_hbm, pol_hbm, act_hbm, obs_hbm, noise_hbm, out_hbm,
             wv, pol_v, act_v, obs_v, noise_v, z2_v, sem_in, sem_out)


def kernel(policies, actions, weights, obs_proc, edge_index):
    del edge_index
    w16 = jnp.broadcast_to(weights.astype(jnp.float32), (16,))
    return _sc_run(w16, policies, actions, obs_proc, _noise_const())
